# Initial kernel scaffold; baseline (speedup 1.0000x reference)
import jax
import jax.numpy as jnp
from jax.experimental import pallas as pl


def kernel(x, edge_index, edge_attr, fW1_1, fb1_1, fW2_1, fb2_1, root_1, bias_1, gamma_1, beta_1, fW1_2, fb1_2, fW2_2, fb2_2, root_2, bias_2, gamma_2, beta_2, fW1_3, fb1_3, fW2_3, fb2_3, root_3, bias_3, gamma_3, beta_3):
    raise NotImplementedError("write your pallas kernel here")


# speedup vs baseline: 2.0168x; 2.0168x over previous
import functools

import jax
import jax.numpy as jnp
from jax import lax
from jax.experimental import pallas as pl
from jax.experimental.pallas import tpu as pltpu
from jax.experimental.pallas import tpu_sc as plsc

N = 10000
E = 40000
DIN = 128
DEMB = 16

NC = 2
NS = 16
NW = NC * NS
CH = 128
EPAD = 40960
CHUNKS_PER_W = EPAD // (NW * CH)
NAGG = 10240
ROWS_PER_SUB = NAGG // NS
TW = 256


def _make_sc_edge_kernel():
    mesh = plsc.VectorSubcoreMesh(core_axis_name="c", subcore_axis_name="s")

    @functools.partial(
        pl.kernel,
        mesh=mesh,
        compiler_params=pltpu.CompilerParams(use_tc_tiling_on_sc=False),
        out_type=jax.ShapeDtypeStruct((NC, NAGG, DEMB), jnp.float32),
        scratch_types=[
            pltpu.VMEM((CH,), jnp.int32),
            pltpu.VMEM((CH,), jnp.int32),
            pltpu.VMEM((CH,), jnp.float32),
            pltpu.VMEM((CH,), jnp.float32),
            pltpu.VMEM((CH, TW), jnp.float32),
            pltpu.VMEM((CH, DEMB), jnp.float32),
            pltpu.VMEM((16,), jnp.float32),
            pltpu.VMEM((16,), jnp.float32),
            pltpu.VMEM((ROWS_PER_SUB, DEMB), jnp.float32),
            pltpu.VMEM_SHARED((NAGG, DEMB), jnp.float32),
            pltpu.SemaphoreType.DMA,
        ],
    )
    def sc_edge(t_hbm, ea_hbm, wg_hbm, src_hbm, dst_hbm, w1_hbm, b1_hbm,
                out_hbm, src_v, dst_v, ea_v, wg_v, rows_v, msg_v,
                w1_v, b1_v, zero_v, agg_sp, sem):
        cid = lax.axis_index("c")
        sid = lax.axis_index("s")
        wid = sid * NC + cid

        def _zrow(i, _):
            zero_v[i, :] = jnp.zeros((16,), jnp.float32)
            return _
        lax.fori_loop(0, ROWS_PER_SUB, _zrow, None)
        pltpu.sync_copy(zero_v, agg_sp.at[pl.ds(sid * ROWS_PER_SUB, ROWS_PER_SUB)])
        plsc.subcore_barrier()

        pltpu.sync_copy(w1_hbm, w1_v)
        pltpu.sync_copy(b1_hbm, b1_v)
        w1 = w1_v[...]
        b1 = b1_v[...]

        def _chunk(ci, _):
            base = (wid * CHUNKS_PER_W + ci) * CH
            pltpu.sync_copy(src_hbm.at[pl.ds(base, CH)], src_v)
            pltpu.sync_copy(dst_hbm.at[pl.ds(base, CH)], dst_v)
            pltpu.sync_copy(ea_hbm.at[pl.ds(base, CH)], ea_v)
            pltpu.sync_copy(wg_hbm.at[pl.ds(base, CH)], wg_v)
            pltpu.async_copy(t_hbm.at[src_v], rows_v, sem).wait()

            def _grp(g, _):
                ea16 = ea_v[pl.ds(g * 16, 16)]
                wg16 = wg_v[pl.ds(g * 16, 16)]
                for j in range(16):
                    ea_s = ea16[j]
                    wg_s = wg16[j]
                    coeff = jnp.maximum(ea_s * w1 + b1, 0.0) * wg_s
                    e = g * 16 + j
                    msg = coeff[0] * rows_v[e, pl.ds(0, 16)]
                    for k in range(1, 16):
                        msg = msg + coeff[k] * rows_v[e, pl.ds(k * 16, 16)]
                    msg_v[e, :] = msg
                return _
            lax.fori_loop(0, CH // 16, _grp, None)

            pltpu.sync_copy(msg_v, agg_sp.at[dst_v], add=True)
            return _
        lax.fori_loop(0, CHUNKS_PER_W, _chunk, None)

        plsc.subcore_barrier()
        r0 = sid * ROWS_PER_SUB
        pltpu.sync_copy(agg_sp.at[pl.ds(r0, ROWS_PER_SUB)],
                        out_hbm.at[cid, pl.ds(r0, ROWS_PER_SUB)])

    return sc_edge


_sc_edge = _make_sc_edge_kernel()


_BR = 2000


def _pre_body(a_ref, c_ref, act_ref, wg_ref, wr_ref, t_ref, r_ref):
    act = act_ref[...] * a_ref[...] + c_ref[...]
    t_ref[...] = jnp.dot(act, wg_ref[...], preferred_element_type=jnp.float32)
    r_ref[...] = jnp.dot(act, wr_ref[...], preferred_element_type=jnp.float32)


def _dense_pre(act, a, c, wgather, wroot):
    din = act.shape[1]
    return pl.pallas_call(
        _pre_body,
        grid=(N // _BR,),
        in_specs=[
            pl.BlockSpec((1, din), lambda i: (0, 0)),
            pl.BlockSpec((1, din), lambda i: (0, 0)),
            pl.BlockSpec((_BR, din), lambda i: (i, 0)),
            pl.BlockSpec((din, TW), lambda i: (0, 0)),
            pl.BlockSpec((din, DEMB), lambda i: (0, 0)),
        ],
        out_specs=[
            pl.BlockSpec((_BR, TW), lambda i: (i, 0)),
            pl.BlockSpec((_BR, DEMB), lambda i: (i, 0)),
        ],
        out_shape=[
            jax.ShapeDtypeStruct((N, TW), jnp.float32),
            jax.ShapeDtypeStruct((N, DEMB), jnp.float32),
        ],
    )(a.reshape(1, din), c.reshape(1, din), act, wgather, wroot)


def _post_body(agg_ref, t_ref, bias_ref, h_ref, st_ref):
    pre = agg_ref[0] + agg_ref[1] + t_ref[...] + bias_ref[...]
    h = jnp.maximum(pre, 0.0)
    h_ref[...] = h

    @pl.when(pl.program_id(0) == 0)
    def _():
        st_ref[...] = jnp.zeros_like(st_ref)

    s = jnp.sum(h, axis=0, keepdims=True)
    ss = jnp.sum(h * h, axis=0, keepdims=True)
    st_ref[...] += jnp.concatenate(
        [s, ss, jnp.zeros((6, DEMB), jnp.float32)], axis=0)


def _dense_post(agg, r, bias):
    return pl.pallas_call(
        _post_body,
        grid=(N // _BR,),
        in_specs=[
            pl.BlockSpec((NC, _BR, DEMB), lambda i: (0, i, 0)),
            pl.BlockSpec((_BR, DEMB), lambda i: (i, 0)),
            pl.BlockSpec((1, DEMB), lambda i: (0, 0)),
        ],
        out_specs=[
            pl.BlockSpec((_BR, DEMB), lambda i: (i, 0)),
            pl.BlockSpec((8, DEMB), lambda i: (0, 0)),
        ],
        out_shape=[
            jax.ShapeDtypeStruct((N, DEMB), jnp.float32),
            jax.ShapeDtypeStruct((8, DEMB), jnp.float32),
        ],
    )(agg, r, bias.reshape(1, DEMB))


def _affine_body(a_ref, c_ref, h_ref, o_ref):
    o_ref[...] = h_ref[...] * a_ref[...] + c_ref[...]


def _affine(h, a, c):
    return pl.pallas_call(
        _affine_body,
        grid=(N // _BR,),
        in_specs=[
            pl.BlockSpec((1, DEMB), lambda i: (0, 0)),
            pl.BlockSpec((1, DEMB), lambda i: (0, 0)),
            pl.BlockSpec((_BR, DEMB), lambda i: (i, 0)),
        ],
        out_specs=pl.BlockSpec((_BR, DEMB), lambda i: (i, 0)),
        out_shape=jax.ShapeDtypeStruct((N, DEMB), jnp.float32),
    )(a.reshape(1, DEMB), c.reshape(1, DEMB), h)


def _pack_wgather(W2, din):
    return W2.reshape(16, din, DEMB).transpose(1, 0, 2).reshape(din, 16 * DEMB)


def _stats_to_affine(st, gamma, beta):
    m = st[0] / N
    v = st[1] / N - m * m
    a = gamma / jnp.sqrt(v + 1e-5)
    c = beta - m * a
    return a, c


def kernel(x, edge_index, edge_attr,
           fW1_1, fb1_1, fW2_1, fb2_1, root_1, bias_1, gamma_1, beta_1,
           fW1_2, fb1_2, fW2_2, fb2_2, root_2, bias_2, gamma_2, beta_2,
           fW1_3, fb1_3, fW2_3, fb2_3, root_3, bias_3, gamma_3, beta_3):
    src = jnp.concatenate(
        [edge_index[0].astype(jnp.int32), jnp.zeros((EPAD - E,), jnp.int32)])
    dst = jnp.concatenate(
        [edge_index[1].astype(jnp.int32), jnp.zeros((EPAD - E,), jnp.int32)])
    ea = jnp.concatenate(
        [edge_attr[:, 0], jnp.zeros((EPAD - E,), jnp.float32)])
    wg = jnp.concatenate(
        [jnp.ones((E,), jnp.float32), jnp.zeros((EPAD - E,), jnp.float32)])

    layers = [
        (fW1_1, fb1_1, fW2_1, fb2_1, root_1, bias_1, gamma_1, beta_1, DIN),
        (fW1_2, fb1_2, fW2_2, fb2_2, root_2, bias_2, gamma_2, beta_2, DEMB),
        (fW1_3, fb1_3, fW2_3, fb2_3, root_3, bias_3, gamma_3, beta_3, DEMB),
    ]

    act = x
    a = jnp.ones((DIN,), jnp.float32)
    c = jnp.zeros((DIN,), jnp.float32)
    for (w1, b1, w2, b2, root, bias, gamma, beta, din) in layers:
        t, r = _dense_pre(act, a, c, _pack_wgather(w2, din), root)
        agg = _sc_edge(t, ea, wg, src, dst, w1.reshape(16), b1)[:, :N, :]
        act, st = _dense_post(agg, r, bias)
        a, c = _stats_to_affine(st, gamma, beta)
    return _affine(act, a, c)
